# Initial kernel scaffold; baseline (speedup 1.0000x reference)
import jax
import jax.numpy as jnp
from jax.experimental import pallas as pl


def kernel(vertex_attr, edgeij_pair, edge_attr, g, batch):
    raise NotImplementedError("write your pallas kernel here")


# speedup vs baseline: 17.1427x; 17.1427x over previous
import functools

import jax
import jax.numpy as jnp
from jax import lax
from jax.experimental import pallas as pl
from jax.experimental.pallas import tpu as pltpu
from jax.experimental.pallas import tpu_sc as plsc

N_V = 100000
N_E = 3200000
NC = 2
NS = 16
NW = NC * NS

EPT = N_E // NW
CHUNK = 10000
NCHUNK = EPT // CHUNK

ACC = 100352
SLICE_SC = ACC // NS
ROWS_PT = ACC // NW
FLAT_PT = ROWS_PT * 4
LAST_ROWS = N_V - (NW - 1) * ROWS_PT
LAST_FLAT = LAST_ROWS * 4

_mesh = plsc.VectorSubcoreMesh(core_axis_name="c", subcore_axis_name="s")


@functools.partial(
    pl.kernel,
    out_type=jax.ShapeDtypeStruct((NC * ACC,), jnp.float32),
    mesh=_mesh,
    scratch_types=[
        pltpu.VMEM((CHUNK,), jnp.int32),
        pltpu.VMEM((CHUNK,), jnp.float32),
        pltpu.VMEM_SHARED((ACC,), jnp.float32),
    ],
)
def _scatter_stage(pair_hbm, ev_hbm, zeros_hbm, out_hbm, idx_v, val_v, acc):
    c = lax.axis_index("c")
    s = lax.axis_index("s")
    w = s * NC + c

    zbase = pl.multiple_of(s * SLICE_SC, 8)
    pltpu.sync_copy(zeros_hbm.at[pl.ds(zbase, SLICE_SC)],
                    acc.at[pl.ds(zbase, SLICE_SC)])
    plsc.subcore_barrier()

    tile_base = pl.multiple_of(w * EPT, 8)
    for k in range(NCHUNK):
        base = tile_base + k * CHUNK
        pltpu.sync_copy(pair_hbm.at[pl.ds(base, CHUNK)], idx_v)
        pltpu.sync_copy(ev_hbm.at[pl.ds(base, CHUNK)], val_v)
        pltpu.sync_copy(val_v, acc.at[idx_v], add=True)
    plsc.subcore_barrier()

    pltpu.sync_copy(acc.at[pl.ds(zbase, SLICE_SC)],
                    out_hbm.at[pl.ds(c * ACC + zbase, SLICE_SC)])


@functools.partial(
    pl.kernel,
    out_type=jax.ShapeDtypeStruct((N_V * 4,), jnp.float32),
    mesh=_mesh,
    scratch_types=[
        pltpu.VMEM((FLAT_PT,), jnp.float32),
        pltpu.VMEM((ROWS_PT,), jnp.float32),
        pltpu.VMEM((ROWS_PT,), jnp.float32),
        pltpu.VMEM((16,), jnp.float32),
    ],
    compiler_params=pltpu.CompilerParams(needs_layout_passes=False),
)
def _update_stage(partials_hbm, vflat_hbm, alpha_hbm, out_hbm,
                  vbuf, p0b, p1b, abuf):
    c = lax.axis_index("c")
    s = lax.axis_index("s")
    w = s * NC + c

    pltpu.sync_copy(alpha_hbm, abuf)
    av = abuf[...]
    iota = lax.iota(jnp.int32, 16)

    vbase = pl.multiple_of(w * ROWS_PT, 8)
    fbase = pl.multiple_of(w * FLAT_PT, 8)
    pltpu.sync_copy(partials_hbm.at[pl.ds(vbase, ROWS_PT)], p0b)
    pltpu.sync_copy(partials_hbm.at[pl.ds(ACC + vbase, ROWS_PT)], p1b)

    def body(nrows, nflat):
        pltpu.sync_copy(vflat_hbm.at[pl.ds(fbase, nflat)],
                        vbuf.at[pl.ds(0, nflat)])

        def step(k, carry):
            i16 = k * 16 + iota
            z = plsc.load_gather(p0b, [i16]) + plsc.load_gather(p1b, [i16])
            ridx = k * 64 + iota * 4 + 2
            r = plsc.load_gather(vbuf, [ridx])
            plsc.store_scatter(vbuf, [ridx], r - av * z)
            return carry

        lax.fori_loop(0, nrows // 16, step, 0)
        pltpu.sync_copy(vbuf.at[pl.ds(0, nflat)],
                        out_hbm.at[pl.ds(fbase, nflat)])

    @pl.when(w < NW - 1)
    def _():
        body(ROWS_PT, FLAT_PT)

    @pl.when(w == NW - 1)
    def _():
        body(LAST_ROWS, LAST_FLAT)


def kernel(vertex_attr, edgeij_pair, edge_attr, g, batch):
    eidx = edgeij_pair.reshape(-1)
    ev = edge_attr.reshape(-1)
    vflat = vertex_attr.reshape(-1)
    alpha16 = jnp.broadcast_to(g[2], (16,)).astype(jnp.float32)
    zeros = jnp.zeros((ACC,), jnp.float32)
    partials = _scatter_stage(eidx, ev, zeros)
    oflat = _update_stage(partials, vflat, alpha16)
    return oflat.reshape(N_V, 4)


# speedup vs baseline: 21.2310x; 1.2385x over previous
import functools

import jax
import jax.numpy as jnp
from jax import lax
from jax.experimental import pallas as pl
from jax.experimental.pallas import tpu as pltpu
from jax.experimental.pallas import tpu_sc as plsc

N_V = 100000
N_E = 3200000
NC = 2
NS = 16
NW = NC * NS

CHUNK = 12544
TILE_E = 8 * CHUNK
FULL_CHUNKS = 7
TAIL_BASE = 255 * CHUNK
TAIL_E = N_E - TAIL_BASE

ACC = 100352
SLICE_SC = ACC // NS
ROWS_PT = ACC // NW
LAST_ROWS = N_V - (NW - 1) * ROWS_PT
SUB_R = 448
LAST_SUBS = LAST_ROWS // SUB_R
LAST_TAIL_R = LAST_ROWS - LAST_SUBS * SUB_R

_mesh = plsc.VectorSubcoreMesh(core_axis_name="c", subcore_axis_name="s")


@functools.partial(
    pl.kernel,
    out_type=jax.ShapeDtypeStruct((NC * ACC,), jnp.float32),
    mesh=_mesh,
    scratch_types=[
        pltpu.VMEM((CHUNK,), jnp.int32),
        pltpu.VMEM((CHUNK,), jnp.float32),
        pltpu.VMEM_SHARED((ACC,), jnp.float32),
    ],
)
def _scatter_stage(eidx_hbm, ev_hbm, zeros_hbm, out_hbm, idx_v, val_v, acc):
    c = lax.axis_index("c")
    s = lax.axis_index("s")
    w = s * NC + c

    zbase = pl.multiple_of(s * SLICE_SC, 8)
    pltpu.sync_copy(zeros_hbm.at[pl.ds(zbase, SLICE_SC)],
                    acc.at[pl.ds(zbase, SLICE_SC)])
    plsc.subcore_barrier()

    def do_chunk(base):
        pltpu.sync_copy(eidx_hbm.at[pl.ds(base, CHUNK)], idx_v)
        pltpu.sync_copy(ev_hbm.at[pl.ds(base, CHUNK)], val_v)
        pltpu.sync_copy(val_v, acc.at[idx_v], add=True)

    tile_base = pl.multiple_of(w * TILE_E, 128)
    for k in range(FULL_CHUNKS):
        do_chunk(tile_base + k * CHUNK)

    @pl.when(w < NW - 1)
    def _():
        do_chunk(tile_base + FULL_CHUNKS * CHUNK)

    @pl.when(w == NW - 1)
    def _():
        pltpu.sync_copy(eidx_hbm.at[pl.ds(TAIL_BASE, TAIL_E)],
                        idx_v.at[pl.ds(0, TAIL_E)])
        pltpu.sync_copy(ev_hbm.at[pl.ds(TAIL_BASE, TAIL_E)],
                        val_v.at[pl.ds(0, TAIL_E)])
        pltpu.sync_copy(zeros_hbm.at[pl.ds(0, CHUNK - TAIL_E)],
                        val_v.at[pl.ds(TAIL_E, CHUNK - TAIL_E)])
        pltpu.sync_copy(val_v, acc.at[idx_v], add=True)

    plsc.subcore_barrier()

    pltpu.sync_copy(acc.at[pl.ds(zbase, SLICE_SC)],
                    out_hbm.at[pl.ds(c * ACC + zbase, SLICE_SC)])


@functools.partial(
    pl.kernel,
    out_type=jax.ShapeDtypeStruct((N_V, 4), jnp.float32),
    mesh=_mesh,
    scratch_types=[
        pltpu.VMEM((SUB_R, 4), jnp.float32),
        pltpu.VMEM((ROWS_PT,), jnp.float32),
        pltpu.VMEM((ROWS_PT,), jnp.float32),
        pltpu.VMEM((16,), jnp.float32),
    ],
    compiler_params=pltpu.CompilerParams(needs_layout_passes=False),
)
def _update_stage(partials_hbm, vattr_hbm, alpha_hbm, out_hbm,
                  vbuf, p0b, p1b, abuf):
    c = lax.axis_index("c")
    s = lax.axis_index("s")
    w = s * NC + c

    pltpu.sync_copy(alpha_hbm, abuf)
    av = abuf[...]
    iota = lax.iota(jnp.int32, 16)
    col2 = jnp.full((16,), 2, jnp.int32)

    vbase = pl.multiple_of(w * ROWS_PT, 8)
    pltpu.sync_copy(partials_hbm.at[pl.ds(vbase, ROWS_PT)], p0b)
    pltpu.sync_copy(partials_hbm.at[pl.ds(ACC + vbase, ROWS_PT)], p1b)

    def subchunk(j, nrows):
        rows0 = vbase + j * SUB_R
        pltpu.sync_copy(vattr_hbm.at[pl.ds(rows0, nrows), :],
                        vbuf.at[pl.ds(0, nrows), :])

        def step(k, carry):
            i16 = k * 16 + iota
            z = (plsc.load_gather(p0b, [j * SUB_R + i16])
                 + plsc.load_gather(p1b, [j * SUB_R + i16]))
            r = plsc.load_gather(vbuf, [i16, col2])
            plsc.store_scatter(vbuf, [i16, col2], r - av * z)
            return carry

        lax.fori_loop(0, nrows // 16, step, 0)
        pltpu.sync_copy(vbuf.at[pl.ds(0, nrows), :],
                        out_hbm.at[pl.ds(rows0, nrows), :])

    @pl.when(w < NW - 1)
    def _():
        for j in range(ROWS_PT // SUB_R):
            subchunk(j, SUB_R)

    @pl.when(w == NW - 1)
    def _():
        for j in range(LAST_SUBS):
            subchunk(j, SUB_R)
        subchunk(LAST_SUBS, LAST_TAIL_R)


def kernel(vertex_attr, edgeij_pair, edge_attr, g, batch):
    eidx = edgeij_pair[0]
    ev = edge_attr.reshape(-1)
    alpha16 = jnp.broadcast_to(g[2], (16,)).astype(jnp.float32)
    zeros = jnp.zeros((ACC,), jnp.float32)
    partials = _scatter_stage(eidx, ev, zeros)
    return _update_stage(partials, vertex_attr, alpha16)
